# Initial kernel scaffold; baseline (speedup 1.0000x reference)
import jax
import jax.numpy as jnp
from jax.experimental import pallas as pl


def kernel(x, weight):
    raise NotImplementedError("write your pallas kernel here")


# speedup vs baseline: 8.5309x; 8.5309x over previous
import functools

import jax
import jax.numpy as jnp
from jax import lax
from jax.experimental import pallas as pl
from jax.experimental.pallas import tpu as pltpu
from jax.experimental.pallas import tpu_sc as plsc

NUM_CORES = 2
NUM_SUBCORES = 16
NUM_WORKERS = NUM_CORES * NUM_SUBCORES
LANES = 16
B = 4096
L = 50
D = 64
ROWS_PER_WORKER = B // NUM_WORKERS
CHUNK = 128


def _make_sc_embedding_bag():
    mesh = plsc.VectorSubcoreMesh(core_axis_name="c", subcore_axis_name="s")

    @functools.partial(
        pl.kernel,
        out_type=jax.ShapeDtypeStruct((B, D), jnp.float32),
        mesh=mesh,
        scratch_types=[
            pltpu.VMEM((L, CHUNK), jnp.int32),
            pltpu.VMEM((CHUNK, D), jnp.float32),
            pltpu.VMEM((CHUNK, D), jnp.float32),
            pltpu.VMEM((ROWS_PER_WORKER, D), jnp.float32),
            pltpu.SemaphoreType.DMA,
            pltpu.SemaphoreType.DMA,
        ],
        compiler_params=pltpu.CompilerParams(use_tc_tiling_on_sc=False),
    )
    def emb_bag(table_hbm, idx_hbm, out_hbm, idx_v, rb0, rb1, acc_v, sem0, sem1):
        wid = lax.axis_index("s") * NUM_CORES + lax.axis_index("c")
        pltpu.sync_copy(idx_hbm.at[wid], idx_v)

        zeros = jnp.zeros((LANES,), jnp.float32)

        @pl.loop(0, ROWS_PER_WORKER)
        def _(r):
            for k in range(D // LANES):
                acc_v[r, pl.ds(k * LANES, LANES)] = zeros

        def start(c, rb, sem):
            pltpu.async_copy(table_hbm.at[idx_v.at[c]], rb, sem)

        def wait(c, rb, sem):
            pltpu.make_async_copy(table_hbm.at[idx_v.at[c]], rb, sem).wait()

        def accum(rb):
            @pl.loop(0, CHUNK)
            def _(r):
                for k in range(D // LANES):
                    sl = pl.ds(k * LANES, LANES)
                    plsc.addupdate(acc_v.at[r, sl], rb[r, sl])

        start(0, rb0, sem0)

        @pl.loop(0, L // 2)
        def _(p):
            c0 = 2 * p
            start(c0 + 1, rb1, sem1)
            wait(c0, rb0, sem0)
            accum(rb0)

            @pl.when(p < L // 2 - 1)
            def _():
                start(c0 + 2, rb0, sem0)

            wait(c0 + 1, rb1, sem1)
            accum(rb1)

        out_slice = out_hbm.at[pl.ds(wid * ROWS_PER_WORKER, ROWS_PER_WORKER)]
        pltpu.sync_copy(acc_v, out_slice)

    return emb_bag


_sc_embedding_bag = _make_sc_embedding_bag()


@jax.jit
def kernel(x, weight):
    idx = x.astype(jnp.int32).reshape(NUM_WORKERS, ROWS_PER_WORKER, L)
    idx = idx.transpose(0, 2, 1)
    return _sc_embedding_bag(weight, idx)


# speedup vs baseline: 9.1219x; 1.0693x over previous
import functools

import jax
import jax.numpy as jnp
from jax import lax
from jax.experimental import pallas as pl
from jax.experimental.pallas import tpu as pltpu
from jax.experimental.pallas import tpu_sc as plsc

NUM_CORES = 2
NUM_SUBCORES = 16
NUM_WORKERS = NUM_CORES * NUM_SUBCORES
LANES = 16
B = 4096
L = 50
D = 64
ROWS_PER_WORKER = B // NUM_WORKERS
CHUNK = 128


def _make_sc_embedding_bag():
    mesh = plsc.VectorSubcoreMesh(core_axis_name="c", subcore_axis_name="s")

    @functools.partial(
        pl.kernel,
        out_type=jax.ShapeDtypeStruct((B, D), jnp.float32),
        mesh=mesh,
        scratch_types=[
            pltpu.VMEM((L, CHUNK), jnp.int32),
            pltpu.VMEM((CHUNK, D), jnp.float32),
            pltpu.VMEM((CHUNK, D), jnp.float32),
            pltpu.VMEM((CHUNK, D), jnp.float32),
            pltpu.VMEM((CHUNK, D), jnp.float32),
            pltpu.VMEM((ROWS_PER_WORKER, D), jnp.float32),
            pltpu.SemaphoreType.DMA,
            pltpu.SemaphoreType.DMA,
            pltpu.SemaphoreType.DMA,
            pltpu.SemaphoreType.DMA,
        ],
        compiler_params=pltpu.CompilerParams(use_tc_tiling_on_sc=False),
    )
    def emb_bag(table_hbm, idx_hbm, out_hbm, idx_v, rb0, rb1, rb2, rb3,
                acc_v, sem0, sem1, sem2, sem3):
        wid = lax.axis_index("s") * NUM_CORES + lax.axis_index("c")
        rbs = (rb0, rb1, rb2, rb3)
        sems = (sem0, sem1, sem2, sem3)
        pltpu.sync_copy(idx_hbm.at[wid], idx_v)

        zeros = jnp.zeros((LANES,), jnp.float32)

        @plsc.parallel_loop(0, ROWS_PER_WORKER)
        def _(r):
            for k in range(D // LANES):
                acc_v[r, pl.ds(k * LANES, LANES)] = zeros

        def start(c, rb, sem):
            pltpu.async_copy(table_hbm.at[idx_v.at[c]], rb, sem)

        def wait(c, rb, sem):
            pltpu.make_async_copy(table_hbm.at[idx_v.at[c]], rb, sem).wait()

        def accum(rb):
            @plsc.parallel_loop(0, CHUNK, unroll=4)
            def _(r):
                for k in range(D // LANES):
                    sl = pl.ds(k * LANES, LANES)
                    plsc.addupdate(acc_v.at[r, sl], rb[r, sl])

        NBUF = 4
        for j in range(NBUF):
            start(j, rbs[j], sems[j])

        @pl.loop(0, L // NBUF)
        def _(p):
            for j in range(NBUF):
                c = NBUF * p + j
                wait(c, rbs[j], sems[j])
                accum(rbs[j])

                @pl.when(c + NBUF < L)
                def _():
                    start(c + NBUF, rbs[j], sems[j])

        for j in range(L % NBUF):
            c = (L // NBUF) * NBUF + j
            wait(c, rbs[j], sems[j])
            accum(rbs[j])

        out_slice = out_hbm.at[pl.ds(wid * ROWS_PER_WORKER, ROWS_PER_WORKER)]
        pltpu.sync_copy(acc_v, out_slice)

    return emb_bag


_sc_embedding_bag = _make_sc_embedding_bag()


@jax.jit
def kernel(x, weight):
    idx = x.astype(jnp.int32).reshape(NUM_WORKERS, ROWS_PER_WORKER, L)
    idx = idx.transpose(0, 2, 1)
    return _sc_embedding_bag(weight, idx)
